# initial kernel scaffold (unmeasured)
import jax
import jax.numpy as jnp
from jax import lax
from jax.experimental import pallas as pl
from jax.experimental.pallas import tpu as pltpu


def kernel(
    x,
):
    def body(*refs):
        pass

    out_shape = jax.ShapeDtypeStruct(..., jnp.float32)
    return pl.pallas_call(body, out_shape=out_shape)(...)


# baseline (device time: 12223 ns/iter reference)
import jax
import jax.numpy as jnp
from jax import lax
from jax.experimental import pallas as pl
from jax.experimental.pallas import tpu as pltpu

M = 512
N = 1024
HALF = N // 2


def kernel(x):
    def body(x_ref, out_ref, send_buf, recv_buf, send_sem, recv_sem):
        my_x = lax.axis_index("x")
        my_y = lax.axis_index("y")
        my_z = lax.axis_index("z")
        other_x = 1 - my_x

        @pl.when(my_x == 0)
        def _():
            send_buf[...] = x_ref[0, :, HALF:].astype(jnp.bfloat16)

        @pl.when(my_x == 1)
        def _():
            send_buf[...] = x_ref[0, :, :HALF].astype(jnp.bfloat16)

        barrier_sem = pltpu.get_barrier_semaphore()
        pl.semaphore_signal(
            barrier_sem,
            inc=1,
            device_id=(other_x, my_y, my_z),
            device_id_type=pl.DeviceIdType.MESH,
        )
        pl.semaphore_wait(barrier_sem, 1)

        rdma = pltpu.make_async_remote_copy(
            src_ref=send_buf,
            dst_ref=recv_buf,
            send_sem=send_sem,
            recv_sem=recv_sem,
            device_id=(other_x, my_y, my_z),
            device_id_type=pl.DeviceIdType.MESH,
        )
        rdma.start()
        rdma.wait()

        @pl.when(my_x == 0)
        def _():
            out_ref[...] = x_ref[0, :, :HALF] + recv_buf[...].astype(jnp.float32)

        @pl.when(my_x == 1)
        def _():
            out_ref[...] = x_ref[0, :, HALF:] + recv_buf[...].astype(jnp.float32)

    return pl.pallas_call(
        body,
        out_shape=jax.ShapeDtypeStruct((M, HALF), jnp.float32),
        in_specs=[pl.BlockSpec(memory_space=pltpu.VMEM)],
        out_specs=pl.BlockSpec(memory_space=pltpu.VMEM),
        scratch_shapes=[
            pltpu.VMEM((M, HALF), jnp.bfloat16),
            pltpu.VMEM((M, HALF), jnp.bfloat16),
            pltpu.SemaphoreType.DMA,
            pltpu.SemaphoreType.DMA,
        ],
        compiler_params=pltpu.CompilerParams(collective_id=0),
    )(x)


# device time: 12146 ns/iter; 1.0063x vs baseline; 1.0063x over previous
import jax
import jax.numpy as jnp
from jax import lax
from jax.experimental import pallas as pl
from jax.experimental.pallas import tpu as pltpu

M = 512
N = 1024
HALF = N // 2
NCHUNK = 4
CROWS = M // NCHUNK


def kernel(x):
    def body(x_ref, out_ref, send_buf, recv_buf, send_sems, recv_sems):
        my_x = lax.axis_index("x")
        my_y = lax.axis_index("y")
        my_z = lax.axis_index("z")
        other_x = 1 - my_x

        def rdma(c):
            return pltpu.make_async_remote_copy(
                src_ref=send_buf.at[pl.ds(c * CROWS, CROWS)],
                dst_ref=recv_buf.at[pl.ds(c * CROWS, CROWS)],
                send_sem=send_sems.at[c],
                recv_sem=recv_sems.at[c],
                device_id=(other_x, my_y, my_z),
                device_id_type=pl.DeviceIdType.MESH,
            )

        barrier_sem = pltpu.get_barrier_semaphore()
        pl.semaphore_signal(
            barrier_sem,
            inc=1,
            device_id=(other_x, my_y, my_z),
            device_id_type=pl.DeviceIdType.MESH,
        )
        pl.semaphore_wait(barrier_sem, 1)

        for c in range(NCHUNK):
            rows = pl.ds(c * CROWS, CROWS)

            @pl.when(my_x == 0)
            def _():
                send_buf[rows] = x_ref[0, rows, HALF:].astype(jnp.bfloat16)

            @pl.when(my_x == 1)
            def _():
                send_buf[rows] = x_ref[0, rows, :HALF].astype(jnp.bfloat16)

            rdma(c).start()

        for c in range(NCHUNK):
            rows = pl.ds(c * CROWS, CROWS)
            rdma(c).wait_recv()

            @pl.when(my_x == 0)
            def _():
                out_ref[rows] = x_ref[0, rows, :HALF] + recv_buf[rows].astype(
                    jnp.float32
                )

            @pl.when(my_x == 1)
            def _():
                out_ref[rows] = x_ref[0, rows, HALF:] + recv_buf[rows].astype(
                    jnp.float32
                )

        for c in range(NCHUNK):
            rdma(c).wait_send()

    return pl.pallas_call(
        body,
        out_shape=jax.ShapeDtypeStruct((M, HALF), jnp.float32),
        in_specs=[pl.BlockSpec(memory_space=pltpu.VMEM)],
        out_specs=pl.BlockSpec(memory_space=pltpu.VMEM),
        scratch_shapes=[
            pltpu.VMEM((M, HALF), jnp.bfloat16),
            pltpu.VMEM((M, HALF), jnp.bfloat16),
            pltpu.SemaphoreType.DMA((NCHUNK,)),
            pltpu.SemaphoreType.DMA((NCHUNK,)),
        ],
        compiler_params=pltpu.CompilerParams(collective_id=0),
    )(x)
